# baseline (speedup 1.0000x reference)
import jax
import jax.numpy as jnp
from jax import lax
from jax.experimental import pallas as pl
from jax.experimental.pallas import tpu as pltpu
from jax.experimental.pallas import tpu_sc as plsc

NC = 2
NS = 16
NW = NC * NS

G = 100
GPC = 2
CH = G * GPC


def _body(seq, emb, nchunk, idx_hbm, tok_hbm, pos_hbm, out_hbm,
          idx_all, tmpl_v, bufs, gsems, wsems):
    wid = lax.axis_index("s") * NC + lax.axis_index("c")
    rbase = wid * (nchunk * GPC)
    obase = wid * (nchunk * CH)

    pltpu.sync_copy(idx_hbm.at[pl.ds(rbase, nchunk * GPC)], idx_all)
    pltpu.sync_copy(pos_hbm, tmpl_v)

    def fire_gathers(u, c):
        for j in range(GPC):
            pltpu.async_copy(tok_hbm.at[idx_all.at[c * GPC + j]],
                             bufs[u].at[pl.ds(j * G, G)], gsems[u])

    def wait_gathers(u):
        for j in range(GPC):
            pltpu.make_async_copy(tok_hbm.at[idx_all.at[0]],
                                  bufs[u].at[pl.ds(j * G, G)],
                                  gsems[u]).wait()

    def fire_wb(u, c):
        off = pl.multiple_of(obase + c * CH, 8)
        pltpu.async_copy(bufs[u], out_hbm.at[pl.ds(off, CH)], wsems[u])

    def wait_wb(u):
        pltpu.make_async_copy(bufs[u], out_hbm.at[pl.ds(0, CH)],
                              wsems[u]).wait()

    def add_pos(u):
        def row_body(r, rcarry):
            for j in range(emb // 16):
                sl = pl.ds(j * 16, 16)
                plsc.addupdate(bufs[u].at[r, sl], tmpl_v[r, sl])
            return rcarry
        lax.fori_loop(0, CH, row_body, 0, unroll=4)

    npair = nchunk // 2

    fire_gathers(0, 0)
    wait_gathers(0)
    fire_gathers(1, 1)
    add_pos(0)
    fire_wb(0, 0)
    wait_gathers(1)
    wait_wb(0)
    fire_gathers(0, 2)
    add_pos(1)
    fire_wb(1, 1)

    def pair_body(k, carry):
        c = k * 2
        wait_gathers(0)
        wait_wb(1)
        fire_gathers(1, c + 1)
        add_pos(0)
        fire_wb(0, c)
        wait_gathers(1)
        wait_wb(0)
        fire_gathers(0, c + 2)
        add_pos(1)
        fire_wb(1, c + 1)
        return carry

    lax.fori_loop(1, npair - 1, pair_body, 0)

    c = nchunk - 2
    wait_gathers(0)
    wait_wb(1)
    fire_gathers(1, c + 1)
    add_pos(0)
    fire_wb(0, c)
    wait_gathers(1)
    wait_wb(0)
    add_pos(1)
    fire_wb(1, c + 1)
    wait_wb(1)


@jax.jit
def kernel(inputs, token_table, pos_table):
    batch, seq = inputs.shape
    emb = token_table.shape[1]
    rows = batch * seq
    nchunk = rows // NW // CH

    idx2 = inputs.astype(jnp.int32).reshape(rows // G, G)
    mesh = plsc.VectorSubcoreMesh(core_axis_name="c", subcore_axis_name="s")
    body = lambda *refs: _body(seq, emb, nchunk, *refs)
    out = pl.kernel(
        body,
        out_type=jax.ShapeDtypeStruct((rows, emb), jnp.float32),
        mesh=mesh,
        compiler_params=pltpu.CompilerParams(use_tc_tiling_on_sc=False),
        scratch_types=[
            pltpu.VMEM((rows // NW // G, G), jnp.int32),
            pltpu.VMEM((seq, emb), jnp.float32),
            [pltpu.VMEM((CH, emb), jnp.float32) for _ in range(2)],
            [pltpu.SemaphoreType.DMA for _ in range(2)],
            [pltpu.SemaphoreType.DMA for _ in range(2)],
        ],
    )(idx2, token_table, pos_table)
    return out.reshape(batch, seq, emb)


# speedup vs baseline: 1.0590x; 1.0590x over previous
import jax
import jax.numpy as jnp
from jax import lax
from jax.experimental import pallas as pl
from jax.experimental.pallas import tpu as pltpu
from jax.experimental.pallas import tpu_sc as plsc

NC = 2
NS = 16
NW = NC * NS

G = 100
GPC = 2
CH = G * GPC


def _body(seq, emb, nchunk, idx_hbm, tok_hbm, pos_hbm, out_hbm,
          idx_all, tmpl_v, bufs, gsems, wsems):
    wid = lax.axis_index("s") * NC + lax.axis_index("c")
    rbase = wid * (nchunk * GPC)
    obase = wid * (nchunk * CH)

    pltpu.sync_copy(idx_hbm.at[pl.ds(rbase, nchunk * GPC)], idx_all)
    pltpu.sync_copy(pos_hbm, tmpl_v)

    def fire_gathers(u, c):
        for j in range(GPC):
            pltpu.async_copy(tok_hbm.at[idx_all.at[c * GPC + j]],
                             bufs[u].at[pl.ds(j * G, G)], gsems[u])

    def wait_gathers(u):
        for j in range(GPC):
            pltpu.make_async_copy(tok_hbm.at[idx_all.at[0]],
                                  bufs[u].at[pl.ds(j * G, G)],
                                  gsems[u]).wait()

    def fire_wb(u, c):
        off = pl.multiple_of(obase + c * CH, 8)
        pltpu.async_copy(bufs[u], out_hbm.at[pl.ds(off, CH)], wsems[u])

    def wait_wb(u):
        pltpu.make_async_copy(bufs[u], out_hbm.at[pl.ds(0, CH)],
                              wsems[u]).wait()

    def add_pos(u):
        def row_body(r, rcarry):
            for j in range(emb // 16):
                sl = pl.ds(j * 16, 16)
                plsc.addupdate(bufs[u].at[r, sl], tmpl_v[r, sl])
            return rcarry
        lax.fori_loop(0, CH, row_body, 0, unroll=4)

    nquad = nchunk // 4

    fire_gathers(0, 0)
    fire_gathers(1, 1)
    wait_gathers(0)
    fire_gathers(2, 2)
    add_pos(0)
    fire_wb(0, 0)
    wait_gathers(1)
    fire_gathers(3, 3)
    add_pos(1)
    fire_wb(1, 1)
    wait_gathers(2)
    wait_wb(0)
    fire_gathers(0, 4)
    add_pos(2)
    fire_wb(2, 2)
    wait_gathers(3)
    wait_wb(1)
    fire_gathers(1, 5)
    add_pos(3)
    fire_wb(3, 3)

    def quad_body(k, carry):
        c = k * 4
        for u in range(4):
            w = (u + 2) % 4
            wait_gathers(u)
            wait_wb(w)
            fire_gathers(w, c + u + 2)
            add_pos(u)
            fire_wb(u, c + u)
        return carry

    lax.fori_loop(1, nquad - 1, quad_body, 0)

    c = nchunk - 4
    wait_gathers(0)
    wait_wb(2)
    fire_gathers(2, c + 2)
    add_pos(0)
    fire_wb(0, c)
    wait_gathers(1)
    wait_wb(3)
    fire_gathers(3, c + 3)
    add_pos(1)
    fire_wb(1, c + 1)
    wait_gathers(2)
    wait_wb(0)
    add_pos(2)
    fire_wb(2, c + 2)
    wait_gathers(3)
    wait_wb(1)
    add_pos(3)
    fire_wb(3, c + 3)
    wait_wb(2)
    wait_wb(3)


@jax.jit
def kernel(inputs, token_table, pos_table):
    batch, seq = inputs.shape
    emb = token_table.shape[1]
    rows = batch * seq
    nchunk = rows // NW // CH

    idx2 = inputs.astype(jnp.int32).reshape(rows // G, G)
    mesh = plsc.VectorSubcoreMesh(core_axis_name="c", subcore_axis_name="s")
    body = lambda *refs: _body(seq, emb, nchunk, *refs)
    out = pl.kernel(
        body,
        out_type=jax.ShapeDtypeStruct((rows, emb), jnp.float32),
        mesh=mesh,
        compiler_params=pltpu.CompilerParams(use_tc_tiling_on_sc=False),
        scratch_types=[
            pltpu.VMEM((rows // NW // G, G), jnp.int32),
            pltpu.VMEM((seq, emb), jnp.float32),
            [pltpu.VMEM((CH, emb), jnp.float32) for _ in range(4)],
            [pltpu.SemaphoreType.DMA for _ in range(4)],
            [pltpu.SemaphoreType.DMA for _ in range(4)],
        ],
    )(idx2, token_table, pos_table)
    return out.reshape(batch, seq, emb)
